# baseline (speedup 1.0000x reference)
import functools

import jax
import jax.numpy as jnp
from jax import lax
from jax.experimental import pallas as pl
from jax.experimental.pallas import tpu as pltpu
from jax.experimental.pallas import tpu_sc as plsc

_BIG = 1e30
_NC, _NS, _L = 2, 16, 16
_NW = _NC * _NS
_GB = 16


def _sc_body(predpack, gtT, gminv_out, pmin_out, pred_v, gt_v, pm_v, gm_v,
             *, B, V1, V2):
    cw = V1 // _NW
    wid = lax.axis_index("s") * _NC + lax.axis_index("c")
    base = wid * cw
    nv = V2 // _L

    def batch_body(b, _):
        pltpu.sync_copy(predpack.at[b], pred_v)
        pltpu.sync_copy(gtT.at[b, :, pl.ds(base, cw)], gt_v)

        for gblk in range(cw // _GB):
            g0 = gblk * _GB
            gxv = gt_v[0, pl.ds(g0, _GB)]
            gyv = gt_v[1, pl.ds(g0, _GB)]
            gzv = gt_v[2, pl.ds(g0, _GB)]
            gx = [jnp.full((_L,), gxv[k], jnp.float32) for k in range(_GB)]
            gy = [jnp.full((_L,), gyv[k], jnp.float32) for k in range(_GB)]
            gz = [jnp.full((_L,), gzv[k], jnp.float32) for k in range(_GB)]

            def vbody(v, gacc, gblk=gblk, gx=gx, gy=gy, gz=gz):
                off = pl.multiple_of(v * _L, _L)
                px = pred_v[0, pl.ds(off, _L)]
                py = pred_v[1, pl.ds(off, _L)]
                pz = pred_v[2, pl.ds(off, _L)]
                bias = pred_v[3, pl.ds(off, _L)]
                news = []
                dmin = None
                for k in range(_GB):
                    dx = px - gx[k]
                    dy = py - gy[k]
                    dz = pz - gz[k]
                    d = dx * dx + dy * dy + dz * dz
                    news.append(jnp.minimum(gacc[k], d + bias))
                    dmin = d if dmin is None else jnp.minimum(dmin, d)
                if gblk == 0:
                    pm_new = dmin
                else:
                    pm_new = jnp.minimum(pm_v[pl.ds(off, _L)], dmin)
                pm_v[pl.ds(off, _L)] = pm_new
                return tuple(news)

            init = tuple(jnp.full((_L,), _BIG, jnp.float32)
                         for _ in range(_GB))
            gacc = lax.fori_loop(0, nv, vbody, init)
            for k in range(_GB):
                gm_v[g0 + k, :] = gacc[k]

        pltpu.sync_copy(gm_v, gminv_out.at[b, pl.ds(base, cw)])
        pltpu.sync_copy(pm_v, pmin_out.at[b, wid])
        return 0

    lax.fori_loop(0, B, batch_body, 0)


def _epi_body(pmin_sc_ref, gminv_ref, m_ref, c_ref, out_p_ref, out_g_ref):
    b = pl.program_id(0)
    pmin = jnp.min(pmin_sc_ref[0], axis=0, keepdims=True)
    m = m_ref[0]
    conf = c_ref[0]
    safe_conf = jnp.where(m > 0, conf, 1.0)
    loss_p = (jnp.maximum(pmin, 0.0) * conf * m
              - jnp.log(safe_conf) * m)
    step_p = jnp.sum(loss_p)
    gmin = jnp.min(gminv_ref[0], axis=1)
    step_g = jnp.sum(jnp.maximum(gmin, 0.0))

    @pl.when(b == 0)
    def _():
        out_p_ref[...] = jnp.zeros_like(out_p_ref)
        out_g_ref[...] = jnp.zeros_like(out_g_ref)

    out_p_ref[...] += jnp.full((1, 1), step_p, jnp.float32)
    out_g_ref[...] += jnp.full((1, 1), step_g, jnp.float32)


def kernel(x_gt, x_pred, mask, confidence):
    B, V1, _ = x_gt.shape
    V2 = x_pred.shape[1]

    m = jnp.squeeze(mask, -1).astype(jnp.float32)
    dbias = (1.0 - m) * _BIG
    predpack = jnp.concatenate(
        [jnp.swapaxes(x_pred, 1, 2), dbias[:, None, :]], axis=1)
    gtT = jnp.swapaxes(x_gt, 1, 2)

    mesh = plsc.VectorSubcoreMesh(core_axis_name="c", subcore_axis_name="s")
    sc = pl.kernel(
        functools.partial(_sc_body, B=B, V1=V1, V2=V2),
        out_type=[
            jax.ShapeDtypeStruct((B, V1, _L), jnp.float32),
            jax.ShapeDtypeStruct((B, _NW, V2), jnp.float32),
        ],
        mesh=mesh,
        scratch_types=[
            pltpu.VMEM((4, V2), jnp.float32),
            pltpu.VMEM((3, V1 // _NW), jnp.float32),
            pltpu.VMEM((V2,), jnp.float32),
            pltpu.VMEM((V1 // _NW, _L), jnp.float32),
        ],
    )
    gminv_sc, pmin_sc = sc(predpack, gtT)

    out_p, out_g = pl.pallas_call(
        _epi_body,
        grid=(B,),
        in_specs=[
            pl.BlockSpec((1, _NW, V2), lambda b: (b, 0, 0)),
            pl.BlockSpec((1, V1, _L), lambda b: (b, 0, 0)),
            pl.BlockSpec((1, 1, V2), lambda b: (b, 0, 0)),
            pl.BlockSpec((1, 1, V2), lambda b: (b, 0, 0)),
        ],
        out_specs=[
            pl.BlockSpec((1, 1), lambda b: (0, 0)),
            pl.BlockSpec((1, 1), lambda b: (0, 0)),
        ],
        out_shape=[
            jax.ShapeDtypeStruct((1, 1), jnp.float32),
            jax.ShapeDtypeStruct((1, 1), jnp.float32),
        ],
    )(pmin_sc, gminv_sc, m[:, None, :], confidence[:, None, :])

    return out_p[0, 0] / (B * V2) + out_g[0, 0] / (B * V1)


# speedup vs baseline: 13.1239x; 13.1239x over previous
import functools

import jax
import jax.numpy as jnp
from jax.experimental import pallas as pl

_BIG = 1e30


def _chamfer_body(g_ref, p_ref, bias_ref, m_ref, c_ref,
                  out_p_ref, out_g_ref, *, num_chunks, tj):
    b = pl.program_id(0)

    P = p_ref[0]
    pn = jax.lax.dot_general(jnp.ones((1, 3), jnp.float32), P * P,
                             (((1,), (1,)), ((), ())),
                             preferred_element_type=jnp.float32)
    pbias = pn + bias_ref[0]

    step_g = jnp.zeros((), jnp.float32)
    pmin = None
    for c in range(num_chunks):
        G = g_ref[0, c * tj:(c + 1) * tj, :]
        gn = jnp.sum(G * G, axis=1, keepdims=True)
        E = jax.lax.dot_general(G * (-2.0), P, (((1,), (1,)), ((), ())),
                                preferred_element_type=jnp.float32)

        gmin = gn + jnp.min(E + pbias, axis=1, keepdims=True)
        step_g += jnp.sum(jnp.maximum(gmin, 0.0))

        cmin = jnp.min(E + gn, axis=0, keepdims=True)
        pmin = cmin if pmin is None else jnp.minimum(pmin, cmin)

    m = m_ref[0]
    conf = c_ref[0]
    safe_conf = jnp.where(m > 0, conf, 1.0)
    loss_p = (jnp.maximum(pmin + pbias, 0.0) * conf * m
              - jnp.log(safe_conf) * m)
    step_p = jnp.sum(loss_p)

    @pl.when(b == 0)
    def _():
        out_p_ref[...] = jnp.zeros_like(out_p_ref)
        out_g_ref[...] = jnp.zeros_like(out_g_ref)

    out_p_ref[...] += jnp.full((1, 1), step_p, jnp.float32)
    out_g_ref[...] += jnp.full((1, 1), step_g, jnp.float32)


def kernel(x_gt, x_pred, mask, confidence):
    B, V1, _ = x_gt.shape
    V2 = x_pred.shape[1]
    TJ = 2048
    num_chunks = V1 // TJ

    m = jnp.squeeze(mask, -1).astype(jnp.float32)
    bias = (1.0 - m) * _BIG

    out_p, out_g = pl.pallas_call(
        functools.partial(_chamfer_body, num_chunks=num_chunks, tj=TJ),
        grid=(B,),
        in_specs=[
            pl.BlockSpec((1, V1, 3), lambda b: (b, 0, 0)),
            pl.BlockSpec((1, V2, 3), lambda b: (b, 0, 0)),
            pl.BlockSpec((1, 1, V2), lambda b: (b, 0, 0)),
            pl.BlockSpec((1, 1, V2), lambda b: (b, 0, 0)),
            pl.BlockSpec((1, 1, V2), lambda b: (b, 0, 0)),
        ],
        out_specs=[
            pl.BlockSpec((1, 1), lambda b: (0, 0)),
            pl.BlockSpec((1, 1), lambda b: (0, 0)),
        ],
        out_shape=[
            jax.ShapeDtypeStruct((1, 1), jnp.float32),
            jax.ShapeDtypeStruct((1, 1), jnp.float32),
        ],
    )(x_gt, x_pred, bias[:, None, :], m[:, None, :], confidence[:, None, :])

    return out_p[0, 0] / (B * V2) + out_g[0, 0] / (B * V1)


# speedup vs baseline: 13.3595x; 1.0179x over previous
import functools

import jax
import jax.numpy as jnp
from jax.experimental import pallas as pl

_BIG = 1e30


def _chamfer_body(g_ref, p_ref, pbias_ref, m_ref, c_ref,
                  out_p_ref, out_g_ref, *, num_chunks, tj):
    b = pl.program_id(0)

    P = p_ref[0]
    pbias = pbias_ref[0]

    step_g = jnp.zeros((), jnp.float32)
    pmin = None
    for c in range(num_chunks):
        G = g_ref[0, c * tj:(c + 1) * tj, :]
        gn = jnp.sum(G * G, axis=1, keepdims=True)
        E = jax.lax.dot_general(G * (-2.0), P, (((1,), (1,)), ((), ())),
                                preferred_element_type=jnp.float32)

        gmin = gn + jnp.min(E + pbias, axis=1, keepdims=True)
        step_g += jnp.sum(jnp.maximum(gmin, 0.0))

        cmin = jnp.min(E + gn, axis=0, keepdims=True)
        pmin = cmin if pmin is None else jnp.minimum(pmin, cmin)

    m = m_ref[0]
    conf = c_ref[0]
    safe_conf = jnp.where(m > 0, conf, 1.0)
    loss_p = (jnp.maximum(pmin + pbias, 0.0) * conf * m
              - jnp.log(safe_conf) * m)
    step_p = jnp.sum(loss_p)

    @pl.when(b == 0)
    def _():
        out_p_ref[...] = jnp.zeros_like(out_p_ref)
        out_g_ref[...] = jnp.zeros_like(out_g_ref)

    out_p_ref[...] += jnp.full((1, 1), step_p, jnp.float32)
    out_g_ref[...] += jnp.full((1, 1), step_g, jnp.float32)


def kernel(x_gt, x_pred, mask, confidence):
    B, V1, _ = x_gt.shape
    V2 = x_pred.shape[1]
    TJ = 1024
    num_chunks = V1 // TJ

    m = jnp.squeeze(mask, -1).astype(jnp.float32)
    pn = jnp.sum(x_pred * x_pred, axis=-1)
    pbias = pn + (1.0 - m) * _BIG

    out_p, out_g = pl.pallas_call(
        functools.partial(_chamfer_body, num_chunks=num_chunks, tj=TJ),
        grid=(B,),
        in_specs=[
            pl.BlockSpec((1, V1, 3), lambda b: (b, 0, 0)),
            pl.BlockSpec((1, V2, 3), lambda b: (b, 0, 0)),
            pl.BlockSpec((1, 1, V2), lambda b: (b, 0, 0)),
            pl.BlockSpec((1, 1, V2), lambda b: (b, 0, 0)),
            pl.BlockSpec((1, 1, V2), lambda b: (b, 0, 0)),
        ],
        out_specs=[
            pl.BlockSpec((1, 1), lambda b: (0, 0)),
            pl.BlockSpec((1, 1), lambda b: (0, 0)),
        ],
        out_shape=[
            jax.ShapeDtypeStruct((1, 1), jnp.float32),
            jax.ShapeDtypeStruct((1, 1), jnp.float32),
        ],
    )(x_gt, x_pred, pbias[:, None, :], m[:, None, :], confidence[:, None, :])

    return out_p[0, 0] / (B * V2) + out_g[0, 0] / (B * V1)


# speedup vs baseline: 13.4595x; 1.0075x over previous
import functools

import jax
import jax.numpy as jnp
from jax.experimental import pallas as pl

_BIG = 1e30


def _chamfer_body(g_ref, p_ref, pbias_ref, m_ref, c_ref,
                  out_p_ref, out_g_ref, *, num_chunks, tj):
    b = pl.program_id(0)

    P = p_ref[0]
    pbias = pbias_ref[0]

    step_g = jnp.zeros((), jnp.float32)
    pmin = None
    for c in range(num_chunks):
        G = g_ref[0, c * tj:(c + 1) * tj, :]
        gn = jnp.sum(G * G, axis=1, keepdims=True)
        E = jax.lax.dot_general(G * (-2.0), P, (((1,), (1,)), ((), ())),
                                preferred_element_type=jnp.float32)

        gmin = gn + jnp.min(E + pbias, axis=1, keepdims=True)
        step_g += jnp.sum(jnp.maximum(gmin, 0.0))

        cmin = jnp.min(E + gn, axis=0, keepdims=True)
        pmin = cmin if pmin is None else jnp.minimum(pmin, cmin)

    m = m_ref[0]
    conf = c_ref[0]
    safe_conf = jnp.where(m > 0, conf, 1.0)
    loss_p = (jnp.maximum(pmin + pbias, 0.0) * conf * m
              - jnp.log(safe_conf) * m)
    step_p = jnp.sum(loss_p)

    @pl.when(b == 0)
    def _():
        out_p_ref[...] = jnp.zeros_like(out_p_ref)
        out_g_ref[...] = jnp.zeros_like(out_g_ref)

    out_p_ref[...] += jnp.full((1, 1), step_p, jnp.float32)
    out_g_ref[...] += jnp.full((1, 1), step_g, jnp.float32)


def kernel(x_gt, x_pred, mask, confidence):
    B, V1, _ = x_gt.shape
    V2 = x_pred.shape[1]
    TJ = 512
    num_chunks = V1 // TJ

    m = jnp.squeeze(mask, -1).astype(jnp.float32)
    pn = jnp.sum(x_pred * x_pred, axis=-1)
    pbias = pn + (1.0 - m) * _BIG

    out_p, out_g = pl.pallas_call(
        functools.partial(_chamfer_body, num_chunks=num_chunks, tj=TJ),
        grid=(B,),
        in_specs=[
            pl.BlockSpec((1, V1, 3), lambda b: (b, 0, 0)),
            pl.BlockSpec((1, V2, 3), lambda b: (b, 0, 0)),
            pl.BlockSpec((1, 1, V2), lambda b: (b, 0, 0)),
            pl.BlockSpec((1, 1, V2), lambda b: (b, 0, 0)),
            pl.BlockSpec((1, 1, V2), lambda b: (b, 0, 0)),
        ],
        out_specs=[
            pl.BlockSpec((1, 1), lambda b: (0, 0)),
            pl.BlockSpec((1, 1), lambda b: (0, 0)),
        ],
        out_shape=[
            jax.ShapeDtypeStruct((1, 1), jnp.float32),
            jax.ShapeDtypeStruct((1, 1), jnp.float32),
        ],
    )(x_gt, x_pred, pbias[:, None, :], m[:, None, :], confidence[:, None, :])

    return out_p[0, 0] / (B * V2) + out_g[0, 0] / (B * V1)
